# baseline (speedup 1.0000x reference)
import functools

import jax
import jax.numpy as jnp
from jax import lax
from jax.experimental import pallas as pl
from jax.experimental.pallas import tpu as pltpu
from jax.experimental.pallas import tpu_sc as plsc

_V = 1000
_D = 1000
_B = 1024 * 50
_NC = 2
_NS = 16
_NW = _NC * _NS
_BPW = _B // _NW
_C = 64
_NCHUNK = _BPW // _C


def _table_body(emb_ref, w_ref, b_ref, t_ref):
    t_ref[...] = lax.dot_general(
        emb_ref[...], w_ref[...],
        dimension_numbers=(((1,), (1,)), ((), ())),
        preferred_element_type=jnp.float32,
    ) + b_ref[...]


_table_call = pl.pallas_call(
    _table_body,
    out_shape=jax.ShapeDtypeStruct((_V, _D), jnp.float32),
)


@functools.partial(
    pl.kernel,
    mesh=plsc.VectorSubcoreMesh(core_axis_name="c", subcore_axis_name="s"),
    out_type=jax.ShapeDtypeStruct((_B, _D), jnp.float32),
    scratch_types=[
        pltpu.VMEM((_NCHUNK, _C), jnp.int32),
        pltpu.VMEM((_C, _D), jnp.float32),
        pltpu.SemaphoreType.DMA,
    ],
    compiler_params=pltpu.CompilerParams(use_tc_tiling_on_sc=False),
)
def _gather_call(table_hbm, idx_hbm, out_hbm, idx_v, buf, sem):
    ci = lax.axis_index("c")
    si = lax.axis_index("s")
    wid = si * _NC + ci
    pltpu.sync_copy(idx_hbm.at[wid], idx_v)

    def chunk(j, carry):
        pltpu.async_copy(table_hbm.at[idx_v.at[j]], buf, sem).wait()
        pltpu.sync_copy(buf, out_hbm.at[pl.ds(wid * _BPW + j * _C, _C)])
        return carry

    lax.fori_loop(0, _NCHUNK, chunk, 0)


def kernel(x, emb_table, W, b):
    table = _table_call(emb_table, W, b.reshape(1, _V))
    idx = x.reshape(_NW, _NCHUNK, _C)
    out = _gather_call(table, idx)
    return out.reshape(x.shape[0], x.shape[1], _V)


# speedup vs baseline: 1.0046x; 1.0046x over previous
import functools

import jax
import jax.numpy as jnp
from jax import lax
from jax.experimental import pallas as pl
from jax.experimental.pallas import tpu as pltpu
from jax.experimental.pallas import tpu_sc as plsc

_V = 1000
_D = 1000
_B = 1024 * 50
_NC = 2
_NS = 16
_NW = _NC * _NS
_BPW = _B // _NW
_C = 64
_NCHUNK = _BPW // _C


def _table_body(emb_ref, w_ref, b_ref, t_ref):
    t_ref[...] = lax.dot_general(
        emb_ref[...], w_ref[...],
        dimension_numbers=(((1,), (1,)), ((), ())),
        preferred_element_type=jnp.float32,
    ) + b_ref[...]


_table_call = pl.pallas_call(
    _table_body,
    out_shape=jax.ShapeDtypeStruct((_V, _D), jnp.float32),
)


@functools.partial(
    pl.kernel,
    mesh=plsc.VectorSubcoreMesh(core_axis_name="c", subcore_axis_name="s"),
    out_type=jax.ShapeDtypeStruct((_B, _D), jnp.float32),
    scratch_types=[
        pltpu.VMEM((_NCHUNK, _C), jnp.int32),
        pltpu.VMEM((_C, _D), jnp.float32),
        pltpu.VMEM((_C, _D), jnp.float32),
        pltpu.SemaphoreType.DMA,
        pltpu.SemaphoreType.DMA,
        pltpu.SemaphoreType.DMA,
        pltpu.SemaphoreType.DMA,
    ],
    compiler_params=pltpu.CompilerParams(use_tc_tiling_on_sc=False),
)
def _gather_call(table_hbm, idx_hbm, out_hbm, idx_v, buf_a, buf_b,
                 sem_ga, sem_gb, sem_sa, sem_sb):
    ci = lax.axis_index("c")
    si = lax.axis_index("s")
    wid = si * _NC + ci
    base = wid * _BPW
    pltpu.sync_copy(idx_hbm.at[wid], idx_v)

    def g_start(j, buf, sem):
        pltpu.async_copy(table_hbm.at[idx_v.at[j]], buf, sem)

    def g_wait(buf, sem):
        pltpu.make_async_copy(table_hbm.at[idx_v.at[0]], buf, sem).wait()

    def s_start(j, buf, sem):
        pltpu.async_copy(buf, out_hbm.at[pl.ds(base + j * _C, _C)], sem)

    def s_wait(buf, sem):
        pltpu.make_async_copy(buf, out_hbm.at[pl.ds(base, _C)], sem).wait()

    g_start(0, buf_a, sem_ga)
    g_wait(buf_a, sem_ga)
    s_start(0, buf_a, sem_sa)
    g_start(1, buf_b, sem_gb)

    def pair(jj, carry):
        c0 = 1 + 2 * jj
        g_wait(buf_b, sem_gb)
        s_wait(buf_a, sem_sa)
        g_start(c0 + 1, buf_a, sem_ga)
        s_start(c0, buf_b, sem_sb)
        g_wait(buf_a, sem_ga)
        s_wait(buf_b, sem_sb)
        g_start(jnp.minimum(c0 + 2, _NCHUNK - 1), buf_b, sem_gb)
        s_start(c0 + 1, buf_a, sem_sa)
        return carry

    lax.fori_loop(0, (_NCHUNK - 1) // 2, pair, 0)
    g_wait(buf_b, sem_gb)
    s_wait(buf_a, sem_sa)


def kernel(x, emb_table, W, b):
    table = _table_call(emb_table, W, b.reshape(1, _V))
    idx = x.reshape(_NW, _NCHUNK, _C)
    out = _gather_call(table, idx)
    return out.reshape(x.shape[0], x.shape[1], _V)


# speedup vs baseline: 1.7063x; 1.6985x over previous
import functools

import jax
import jax.numpy as jnp
from jax import lax
from jax.experimental import pallas as pl
from jax.experimental.pallas import tpu as pltpu
from jax.experimental.pallas import tpu_sc as plsc

_V = 1000
_D = 16
_DP = 128
_BATCH = 1024
_SEQ = 50
_B = _BATCH * _SEQ
_NC = 2
_NS = 16
_NW = _NC * _NS
_TPW = _B // _NW
_C = 80
_NCHUNK = _TPW // _C

_BB = 8
_TB = _BB * _SEQ


@functools.partial(
    pl.kernel,
    mesh=plsc.VectorSubcoreMesh(core_axis_name="c", subcore_axis_name="s"),
    out_type=jax.ShapeDtypeStruct((_B, _DP), jnp.float32),
    scratch_types=[
        pltpu.VMEM((_NCHUNK, _C), jnp.int32),
        pltpu.VMEM((_C, _DP), jnp.float32),
        pltpu.VMEM((_C, _DP), jnp.float32),
        pltpu.SemaphoreType.DMA,
        pltpu.SemaphoreType.DMA,
        pltpu.SemaphoreType.DMA,
        pltpu.SemaphoreType.DMA,
    ],
)
def _sc_lookup(table_hbm, idx_hbm, out_hbm, idx_v, buf_a, buf_b,
               sem_ga, sem_gb, sem_sa, sem_sb):
    ci = lax.axis_index("c")
    si = lax.axis_index("s")
    wid = si * _NC + ci
    base = wid * _TPW
    pltpu.sync_copy(idx_hbm.at[wid], idx_v)

    def g_start(j, buf, sem):
        pltpu.async_copy(table_hbm.at[idx_v.at[j]], buf, sem)

    def g_wait(buf, sem):
        pltpu.make_async_copy(table_hbm.at[idx_v.at[0]], buf, sem).wait()

    def s_start(j, buf, sem):
        pltpu.async_copy(buf, out_hbm.at[pl.ds(base + j * _C, _C)], sem)

    def s_wait(buf, sem):
        pltpu.make_async_copy(buf, out_hbm.at[pl.ds(base, _C)], sem).wait()

    g_start(0, buf_a, sem_ga)
    g_wait(buf_a, sem_ga)
    s_start(0, buf_a, sem_sa)
    g_start(1, buf_b, sem_gb)

    def pair(jj, carry):
        c0 = 1 + 2 * jj
        g_wait(buf_b, sem_gb)
        s_wait(buf_a, sem_sa)
        g_start(c0 + 1, buf_a, sem_ga)
        s_start(c0, buf_b, sem_sb)
        g_wait(buf_a, sem_ga)
        s_wait(buf_b, sem_sb)
        g_start(jnp.minimum(c0 + 2, _NCHUNK - 1), buf_b, sem_gb)
        s_start(c0 + 1, buf_a, sem_sa)
        return carry

    lax.fori_loop(0, (_NCHUNK - 1) // 2, pair, 0)
    g_wait(buf_b, sem_gb)
    s_wait(buf_a, sem_sa)


def _proj_body(h_ref, w_ref, b_ref, out_ref):
    hb = h_ref[:, : _D]
    acc = lax.dot_general(
        hb, w_ref[...],
        dimension_numbers=(((1,), (1,)), ((), ())),
        preferred_element_type=jnp.float32,
    ) + b_ref[...]
    out_ref[...] = acc.reshape(_BB, _SEQ, _V)


_proj_call = pl.pallas_call(
    _proj_body,
    grid=(_B // _TB,),
    in_specs=[
        pl.BlockSpec((_TB, _DP), lambda i: (i, 0)),
        pl.BlockSpec((_V, _D), lambda i: (0, 0)),
        pl.BlockSpec((1, _V), lambda i: (0, 0)),
    ],
    out_specs=pl.BlockSpec((_BB, _SEQ, _V), lambda i: (i, 0, 0)),
    out_shape=jax.ShapeDtypeStruct((_BATCH, _SEQ, _V), jnp.float32),
)


def kernel(x, emb_table, W, b):
    table = jnp.pad(emb_table, ((0, 0), (0, _DP - _D)))
    idx = x.reshape(_NW, _NCHUNK, _C)
    h = _sc_lookup(table, idx)
    return _proj_call(h, W, b.reshape(1, _V))


# speedup vs baseline: 1.7232x; 1.0099x over previous
import functools

import jax
import jax.numpy as jnp
from jax import lax
from jax.experimental import pallas as pl
from jax.experimental.pallas import tpu as pltpu
from jax.experimental.pallas import tpu_sc as plsc

_V = 1000
_D = 16
_DP = 128
_BATCH = 1024
_SEQ = 50
_B = _BATCH * _SEQ
_NC = 2
_NS = 16
_NW = _NC * _NS
_TPW = _B // _NW
_C = 80
_NCHUNK = _TPW // _C

_BB = 8
_TB = _BB * _SEQ


@functools.partial(
    pl.kernel,
    mesh=plsc.VectorSubcoreMesh(core_axis_name="c", subcore_axis_name="s"),
    out_type=jax.ShapeDtypeStruct((_B, _DP), jnp.float32),
    scratch_types=[
        pltpu.VMEM((_TPW,), jnp.int32),
        pltpu.VMEM((_C, _DP), jnp.float32),
        pltpu.VMEM((_C, _DP), jnp.float32),
        pltpu.SemaphoreType.DMA,
        pltpu.SemaphoreType.DMA,
        pltpu.SemaphoreType.DMA,
        pltpu.SemaphoreType.DMA,
    ],
)
def _sc_lookup(table_hbm, idx_hbm, out_hbm, idx_v, buf_a, buf_b,
               sem_ga, sem_gb, sem_sa, sem_sb):
    ci = lax.axis_index("c")
    si = lax.axis_index("s")
    wid = si * _NC + ci
    base = wid * _TPW
    pltpu.sync_copy(idx_hbm.at[pl.ds(base, _TPW)], idx_v)

    def g_start(j, buf, sem):
        pltpu.async_copy(table_hbm.at[idx_v.at[pl.ds(j * _C, _C)]], buf, sem)

    def g_wait(buf, sem):
        pltpu.make_async_copy(table_hbm.at[idx_v.at[pl.ds(0, _C)]], buf, sem).wait()

    def s_start(j, buf, sem):
        pltpu.async_copy(buf, out_hbm.at[pl.ds(base + j * _C, _C)], sem)

    def s_wait(buf, sem):
        pltpu.make_async_copy(buf, out_hbm.at[pl.ds(base, _C)], sem).wait()

    g_start(0, buf_a, sem_ga)
    g_wait(buf_a, sem_ga)
    s_start(0, buf_a, sem_sa)
    g_start(1, buf_b, sem_gb)

    def pair(jj, carry):
        c0 = 1 + 2 * jj
        g_wait(buf_b, sem_gb)
        s_wait(buf_a, sem_sa)
        g_start(c0 + 1, buf_a, sem_ga)
        s_start(c0, buf_b, sem_sb)
        g_wait(buf_a, sem_ga)
        s_wait(buf_b, sem_sb)
        g_start(jnp.minimum(c0 + 2, _NCHUNK - 1), buf_b, sem_gb)
        s_start(c0 + 1, buf_a, sem_sa)
        return carry

    lax.fori_loop(0, (_NCHUNK - 1) // 2, pair, 0)
    g_wait(buf_b, sem_gb)
    s_wait(buf_a, sem_sa)
    s_start(_NCHUNK - 1, buf_b, sem_sb)
    s_wait(buf_b, sem_sb)


def _proj_body(h_ref, w_ref, b_ref, out_ref):
    hb = h_ref[:, : _D]
    acc = lax.dot_general(
        hb, w_ref[...],
        dimension_numbers=(((1,), (1,)), ((), ())),
        preferred_element_type=jnp.float32,
    ) + b_ref[...]
    out_ref[...] = acc.reshape(_BB, _SEQ, _V)


_proj_call = pl.pallas_call(
    _proj_body,
    grid=(_B // _TB,),
    in_specs=[
        pl.BlockSpec((_TB, _DP), lambda i: (i, 0)),
        pl.BlockSpec((_V, _D), lambda i: (0, 0)),
        pl.BlockSpec((1, _V), lambda i: (0, 0)),
    ],
    out_specs=pl.BlockSpec((_BB, _SEQ, _V), lambda i: (i, 0, 0)),
    out_shape=jax.ShapeDtypeStruct((_BATCH, _SEQ, _V), jnp.float32),
)


def kernel(x, emb_table, W, b):
    table = jnp.pad(emb_table, ((0, 0), (0, _DP - _D)))
    idx = x.reshape(_B)
    h = _sc_lookup(table, idx)
    return _proj_call(h, W, b.reshape(1, _V))
